# baseline (speedup 1.0000x reference)
import functools

import jax
import jax.numpy as jnp
from jax import lax
from jax.experimental import pallas as pl
from jax.experimental.pallas import tpu as pltpu
from jax.experimental.pallas import tpu_sc as plsc

BATCH = 16384
DIM = 64
NUM_WORKERS = 32
ROWS_PER_WORKER = BATCH // NUM_WORKERS
CHUNK = 128
NUM_CHUNKS = ROWS_PER_WORKER // CHUNK
LANES = 16
DIM_CHUNKS = DIM // LANES


def _body(heads_hbm, rels_hbm, tails_hbm, er_hbm, ei_hbm, rr_hbm, ri_hbm,
          out_hbm,
          idx_h, idx_r, idx_t,
          hr_v, hi_v, tr_v, ti_v, rr_v, ri_v,
          tmp_v, out_v, sem):
    wid = lax.axis_index("s") * 2 + lax.axis_index("c")
    base = wid * ROWS_PER_WORKER

    pltpu.sync_copy(heads_hbm.at[pl.ds(base, ROWS_PER_WORKER)], idx_h)
    pltpu.sync_copy(rels_hbm.at[pl.ds(base, ROWS_PER_WORKER)], idx_r)
    pltpu.sync_copy(tails_hbm.at[pl.ds(base, ROWS_PER_WORKER)], idx_t)

    for k in range(NUM_CHUNKS):
        off = k * CHUNK
        ih = idx_h.at[pl.ds(off, CHUNK)]
        ir = idx_r.at[pl.ds(off, CHUNK)]
        it = idx_t.at[pl.ds(off, CHUNK)]
        copies = [
            pltpu.async_copy(er_hbm.at[ih], hr_v, sem),
            pltpu.async_copy(ei_hbm.at[ih], hi_v, sem),
            pltpu.async_copy(er_hbm.at[it], tr_v, sem),
            pltpu.async_copy(ei_hbm.at[it], ti_v, sem),
            pltpu.async_copy(rr_hbm.at[ir], rr_v, sem),
            pltpu.async_copy(ri_hbm.at[ir], ri_v, sem),
        ]
        for c in copies:
            c.wait()

        lane = lax.iota(jnp.int32, LANES)

        def group_body(g, carry):
            row0 = g * LANES
            for j in range(LANES):
                r = row0 + j
                acc = jnp.zeros((LANES,), jnp.float32)
                for d in range(DIM_CHUNKS):
                    sl = pl.ds(d * LANES, LANES)
                    hr = hr_v[r, sl]
                    hi = hi_v[r, sl]
                    tr = tr_v[r, sl]
                    ti = ti_v[r, sl]
                    rr = rr_v[r, sl]
                    ri = ri_v[r, sl]
                    acc = (acc + tr * (hr * rr + hi * ri)
                           + ti * (hr * ri - hi * rr))
                tmp_v[j, :] = acc
            scores = jnp.zeros((LANES,), jnp.float32)
            for k in range(LANES):
                col = jnp.full((LANES,), k, jnp.int32)
                scores = scores + plsc.load_gather(tmp_v, [lane, col])
            out_v[pl.ds(off + row0, LANES)] = scores
            return carry

        lax.fori_loop(0, CHUNK // LANES, group_body, 0)

    pltpu.sync_copy(out_v, out_hbm.at[pl.ds(base, ROWS_PER_WORKER)])


@jax.jit
def _complex_score(heads, relations, tails, entity_real, entity_imag,
                   relation_real, relation_imag):
    mesh = plsc.VectorSubcoreMesh(core_axis_name="c", subcore_axis_name="s")
    kern = pl.kernel(
        _body,
        out_type=jax.ShapeDtypeStruct((BATCH,), jnp.float32),
        mesh=mesh,
        compiler_params=pltpu.CompilerParams(needs_layout_passes=False,
                                             use_tc_tiling_on_sc=False),
        scratch_types=[
            pltpu.VMEM((ROWS_PER_WORKER,), jnp.int32),
            pltpu.VMEM((ROWS_PER_WORKER,), jnp.int32),
            pltpu.VMEM((ROWS_PER_WORKER,), jnp.int32),
            pltpu.VMEM((CHUNK, DIM), jnp.float32),
            pltpu.VMEM((CHUNK, DIM), jnp.float32),
            pltpu.VMEM((CHUNK, DIM), jnp.float32),
            pltpu.VMEM((CHUNK, DIM), jnp.float32),
            pltpu.VMEM((CHUNK, DIM), jnp.float32),
            pltpu.VMEM((CHUNK, DIM), jnp.float32),
            pltpu.VMEM((LANES, LANES), jnp.float32),
            pltpu.VMEM((ROWS_PER_WORKER,), jnp.float32),
            pltpu.SemaphoreType.DMA,
        ],
    )
    return kern(heads, relations, tails, entity_real, entity_imag,
                relation_real, relation_imag)


def kernel(heads, relations, tails, entity_real, entity_imag,
           relation_real, relation_imag):
    return _complex_score(heads.astype(jnp.int32), relations.astype(jnp.int32),
                          tails.astype(jnp.int32), entity_real, entity_imag,
                          relation_real, relation_imag)


# speedup vs baseline: 1.4899x; 1.4899x over previous
import functools

import jax
import jax.numpy as jnp
from jax import lax
from jax.experimental import pallas as pl
from jax.experimental.pallas import tpu as pltpu
from jax.experimental.pallas import tpu_sc as plsc

BATCH = 16384
DIM = 64
NUM_WORKERS = 32
ROWS_PER_WORKER = BATCH // NUM_WORKERS
LANES = 16
DIM_CHUNKS = DIM // LANES
NUM_GROUPS = ROWS_PER_WORKER // LANES


def _body(heads_hbm, rels_hbm, tails_hbm, er_hbm, ei_hbm, rr_hbm, ri_hbm,
          out_hbm,
          idx_h, idx_r, idx_t,
          hr_v, hi_v, tr_v, ti_v, rr_v, ri_v,
          tmp_v, out_v, sem):
    wid = lax.axis_index("s") * 2 + lax.axis_index("c")
    base = wid * ROWS_PER_WORKER

    pltpu.sync_copy(heads_hbm.at[pl.ds(base, ROWS_PER_WORKER)], idx_h)
    pltpu.sync_copy(rels_hbm.at[pl.ds(base, ROWS_PER_WORKER)], idx_r)
    pltpu.sync_copy(tails_hbm.at[pl.ds(base, ROWS_PER_WORKER)], idx_t)

    lane = lax.iota(jnp.int32, LANES)

    def group_body(g, carry):
        off = g * LANES
        vh = idx_h[pl.ds(off, LANES)]
        vt = idx_t[pl.ds(off, LANES)]
        vr = idx_r[pl.ds(off, LANES)]
        copies = []
        for j in range(LANES):
            h = vh[j]
            t = vt[j]
            r = vr[j]
            copies.append(pltpu.async_copy(er_hbm.at[h], hr_v.at[j], sem))
            copies.append(pltpu.async_copy(ei_hbm.at[h], hi_v.at[j], sem))
            copies.append(pltpu.async_copy(er_hbm.at[t], tr_v.at[j], sem))
            copies.append(pltpu.async_copy(ei_hbm.at[t], ti_v.at[j], sem))
            copies.append(pltpu.async_copy(rr_hbm.at[r], rr_v.at[j], sem))
            copies.append(pltpu.async_copy(ri_hbm.at[r], ri_v.at[j], sem))
        for c in copies:
            c.wait()

        for j in range(LANES):
            acc = jnp.zeros((LANES,), jnp.float32)
            for d in range(DIM_CHUNKS):
                sl = pl.ds(d * LANES, LANES)
                hr = hr_v[j, sl]
                hi = hi_v[j, sl]
                tr = tr_v[j, sl]
                ti = ti_v[j, sl]
                rr = rr_v[j, sl]
                ri = ri_v[j, sl]
                acc = (acc + tr * (hr * rr + hi * ri)
                       + ti * (hr * ri - hi * rr))
            tmp_v[j, :] = acc
        scores = jnp.zeros((LANES,), jnp.float32)
        for k in range(LANES):
            col = jnp.full((LANES,), k, jnp.int32)
            scores = scores + plsc.load_gather(tmp_v, [lane, col])
        out_v[pl.ds(off, LANES)] = scores
        return carry

    lax.fori_loop(0, NUM_GROUPS, group_body, 0)

    pltpu.sync_copy(out_v, out_hbm.at[pl.ds(base, ROWS_PER_WORKER)])


@jax.jit
def _complex_score(heads, relations, tails, entity_real, entity_imag,
                   relation_real, relation_imag):
    mesh = plsc.VectorSubcoreMesh(core_axis_name="c", subcore_axis_name="s")
    kern = pl.kernel(
        _body,
        out_type=jax.ShapeDtypeStruct((BATCH,), jnp.float32),
        mesh=mesh,
        compiler_params=pltpu.CompilerParams(needs_layout_passes=False),
        scratch_types=[
            pltpu.VMEM((ROWS_PER_WORKER,), jnp.int32),
            pltpu.VMEM((ROWS_PER_WORKER,), jnp.int32),
            pltpu.VMEM((ROWS_PER_WORKER,), jnp.int32),
            pltpu.VMEM((LANES, DIM), jnp.float32),
            pltpu.VMEM((LANES, DIM), jnp.float32),
            pltpu.VMEM((LANES, DIM), jnp.float32),
            pltpu.VMEM((LANES, DIM), jnp.float32),
            pltpu.VMEM((LANES, DIM), jnp.float32),
            pltpu.VMEM((LANES, DIM), jnp.float32),
            pltpu.VMEM((LANES, LANES), jnp.float32),
            pltpu.VMEM((ROWS_PER_WORKER,), jnp.float32),
            pltpu.SemaphoreType.DMA,
        ],
    )
    return kern(heads, relations, tails, entity_real, entity_imag,
                relation_real, relation_imag)


def kernel(heads, relations, tails, entity_real, entity_imag,
           relation_real, relation_imag):
    return _complex_score(heads.astype(jnp.int32), relations.astype(jnp.int32),
                          tails.astype(jnp.int32), entity_real, entity_imag,
                          relation_real, relation_imag)


# speedup vs baseline: 1.4922x; 1.0015x over previous
import functools

import jax
import jax.numpy as jnp
from jax import lax
from jax.experimental import pallas as pl
from jax.experimental.pallas import tpu as pltpu
from jax.experimental.pallas import tpu_sc as plsc

BATCH = 16384
DIM = 64
NUM_WORKERS = 32
ROWS_PER_WORKER = BATCH // NUM_WORKERS
LANES = 16
DIM_CHUNKS = DIM // LANES
NUM_GROUPS = ROWS_PER_WORKER // LANES


def _body(heads_hbm, rels_hbm, tails_hbm, er_hbm, ei_hbm, rr_hbm, ri_hbm,
          out_hbm,
          idx_h, idx_r, idx_t,
          hr_v, hi_v, tr_v, ti_v, rr_v, ri_v,
          tmp_v, out_v,
          sem_hr, sem_hi, sem_tr, sem_ti, sem_rr, sem_ri):
    wid = lax.axis_index("s") * 2 + lax.axis_index("c")
    base = wid * ROWS_PER_WORKER

    pltpu.sync_copy(heads_hbm.at[pl.ds(base, ROWS_PER_WORKER)], idx_h)
    pltpu.sync_copy(rels_hbm.at[pl.ds(base, ROWS_PER_WORKER)], idx_r)
    pltpu.sync_copy(tails_hbm.at[pl.ds(base, ROWS_PER_WORKER)], idx_t)

    lane = lax.iota(jnp.int32, LANES)

    def group_body(g, carry):
        off = g * LANES
        vh = idx_h[pl.ds(off, LANES)]
        vt = idx_t[pl.ds(off, LANES)]
        vr = idx_r[pl.ds(off, LANES)]
        copies = []
        for j in range(LANES):
            h = vh[j]
            t = vt[j]
            r = vr[j]
            copies.append(pltpu.async_copy(er_hbm.at[h], hr_v.at[j], sem_hr))
            copies.append(pltpu.async_copy(ei_hbm.at[h], hi_v.at[j], sem_hi))
            copies.append(pltpu.async_copy(er_hbm.at[t], tr_v.at[j], sem_tr))
            copies.append(pltpu.async_copy(ei_hbm.at[t], ti_v.at[j], sem_ti))
            copies.append(pltpu.async_copy(rr_hbm.at[r], rr_v.at[j], sem_rr))
            copies.append(pltpu.async_copy(ri_hbm.at[r], ri_v.at[j], sem_ri))
        for c in copies:
            c.wait()

        for j in range(LANES):
            acc = jnp.zeros((LANES,), jnp.float32)
            for d in range(DIM_CHUNKS):
                sl = pl.ds(d * LANES, LANES)
                hr = hr_v[j, sl]
                hi = hi_v[j, sl]
                tr = tr_v[j, sl]
                ti = ti_v[j, sl]
                rr = rr_v[j, sl]
                ri = ri_v[j, sl]
                acc = (acc + tr * (hr * rr + hi * ri)
                       + ti * (hr * ri - hi * rr))
            tmp_v[j, :] = acc
        scores = jnp.zeros((LANES,), jnp.float32)
        for k in range(LANES):
            col = jnp.full((LANES,), k, jnp.int32)
            scores = scores + plsc.load_gather(tmp_v, [lane, col])
        out_v[pl.ds(off, LANES)] = scores
        return carry

    lax.fori_loop(0, NUM_GROUPS, group_body, 0)

    pltpu.sync_copy(out_v, out_hbm.at[pl.ds(base, ROWS_PER_WORKER)])


@jax.jit
def _complex_score(heads, relations, tails, entity_real, entity_imag,
                   relation_real, relation_imag):
    mesh = plsc.VectorSubcoreMesh(core_axis_name="c", subcore_axis_name="s")
    kern = pl.kernel(
        _body,
        out_type=jax.ShapeDtypeStruct((BATCH,), jnp.float32),
        mesh=mesh,
        compiler_params=pltpu.CompilerParams(needs_layout_passes=False),
        scratch_types=[
            pltpu.VMEM((ROWS_PER_WORKER,), jnp.int32),
            pltpu.VMEM((ROWS_PER_WORKER,), jnp.int32),
            pltpu.VMEM((ROWS_PER_WORKER,), jnp.int32),
            pltpu.VMEM((LANES, DIM), jnp.float32),
            pltpu.VMEM((LANES, DIM), jnp.float32),
            pltpu.VMEM((LANES, DIM), jnp.float32),
            pltpu.VMEM((LANES, DIM), jnp.float32),
            pltpu.VMEM((LANES, DIM), jnp.float32),
            pltpu.VMEM((LANES, DIM), jnp.float32),
            pltpu.VMEM((LANES, LANES), jnp.float32),
            pltpu.VMEM((ROWS_PER_WORKER,), jnp.float32),
            pltpu.SemaphoreType.DMA,
            pltpu.SemaphoreType.DMA,
            pltpu.SemaphoreType.DMA,
            pltpu.SemaphoreType.DMA,
            pltpu.SemaphoreType.DMA,
            pltpu.SemaphoreType.DMA,
        ],
    )
    return kern(heads, relations, tails, entity_real, entity_imag,
                relation_real, relation_imag)


def kernel(heads, relations, tails, entity_real, entity_imag,
           relation_real, relation_imag):
    return _complex_score(heads.astype(jnp.int32), relations.astype(jnp.int32),
                          tails.astype(jnp.int32), entity_real, entity_imag,
                          relation_real, relation_imag)
